# baseline (speedup 1.0000x reference)
import functools

import jax
import jax.numpy as jnp
from jax import lax
from jax.experimental import pallas as pl
from jax.experimental.pallas import tpu as pltpu
from jax.experimental.pallas import tpu_sc as plsc

S, D, F, ER, TOPK = 2048, 1024, 1024, 7, 2
A = S * TOPK
BM = 256
NB = 24
M_PAD = NB * BM
LANES = 128

_NC, _NS = 2, 16
_NW = _NC * _NS


def _routing_body(x_ref, wr_ref, b_ref, ei_ref, wv_ref):
    x = x_ref[...]
    logits = jnp.dot(x, wr_ref[...], preferred_element_type=jnp.float32)
    logits = logits + b_ref[...]
    lane = lax.broadcasted_iota(jnp.int32, (S, LANES), 1)
    probs = jax.nn.sigmoid(logits)
    probs = jnp.where(lane < ER, probs, -1.0)
    m1 = jnp.max(probs, axis=1, keepdims=True)
    i1 = jnp.min(jnp.where(probs == m1, lane, LANES), axis=1, keepdims=True)
    probs2 = jnp.where(lane == i1, -1.0, probs)
    m2 = jnp.max(probs2, axis=1, keepdims=True)
    i2 = jnp.min(jnp.where(probs2 == m2, lane, LANES), axis=1, keepdims=True)
    ssum = m1 + m2
    ei_ref[...] = jnp.where(lane == 0, i1, jnp.where(lane == 1, i2, 0))
    wv_ref[...] = jnp.where(lane == 0, m1 / ssum,
                            jnp.where(lane == 1, m2 / ssum, 0.0))


def _routing(x2, wr_pad, b_pad):
    return pl.pallas_call(
        _routing_body,
        out_shape=(
            jax.ShapeDtypeStruct((S, LANES), jnp.int32),
            jax.ShapeDtypeStruct((S, LANES), jnp.float32),
        ),
    )(x2, wr_pad, b_pad)


def _grouped_body(sc_ref, xg_ref, wg_ref, wu_ref, wd_ref, wp_ref, y_ref):
    i = pl.program_id(0)

    @pl.when(i < sc_ref[NB])
    def _():
        x = xg_ref[...].astype(jnp.bfloat16)
        g = jnp.dot(x, wg_ref[0].astype(jnp.bfloat16),
                    preferred_element_type=jnp.float32)
        u = jnp.dot(x, wu_ref[0].astype(jnp.bfloat16),
                    preferred_element_type=jnp.float32)
        h = g * jax.nn.sigmoid(g) * u * wp_ref[:, :1]
        y_ref[...] = jnp.dot(h.astype(jnp.bfloat16),
                             wd_ref[0].astype(jnp.bfloat16),
                             preferred_element_type=jnp.float32)


def _grouped(scalars, xg, Wg, Wu, Wd, w2d):
    grid_spec = pltpu.PrefetchScalarGridSpec(
        num_scalar_prefetch=1,
        grid=(NB,),
        in_specs=[
            pl.BlockSpec((BM, D), lambda i, sc: (i, 0)),
            pl.BlockSpec((1, D, F), lambda i, sc: (sc[i], 0, 0)),
            pl.BlockSpec((1, D, F), lambda i, sc: (sc[i], 0, 0)),
            pl.BlockSpec((1, F, D), lambda i, sc: (sc[i], 0, 0)),
            pl.BlockSpec((BM, LANES), lambda i, sc: (i, 0)),
        ],
        out_specs=pl.BlockSpec((BM, D), lambda i, sc: (i, 0)),
    )
    return pl.pallas_call(
        _grouped_body,
        grid_spec=grid_spec,
        out_shape=jax.ShapeDtypeStruct((M_PAD, D), jnp.float32),
    )(scalars, xg, Wg, Wu, Wd, w2d)


def _shared_body(x_ref, g_ref, u_ref, d_ref, o_ref):
    x = x_ref[...].astype(jnp.bfloat16)
    g = jnp.dot(x, g_ref[...].astype(jnp.bfloat16),
                preferred_element_type=jnp.float32)
    u = jnp.dot(x, u_ref[...].astype(jnp.bfloat16),
                preferred_element_type=jnp.float32)
    h = g * jax.nn.sigmoid(g) * u
    o_ref[...] = jnp.dot(h.astype(jnp.bfloat16), d_ref[...].astype(jnp.bfloat16),
                         preferred_element_type=jnp.float32)


def _shared(x2, gate_s, up_s, down_s):
    bm = 256
    return pl.pallas_call(
        _shared_body,
        grid=(S // bm,),
        in_specs=[
            pl.BlockSpec((bm, D), lambda i: (i, 0)),
            pl.BlockSpec((D, F), lambda i: (0, 0)),
            pl.BlockSpec((D, F), lambda i: (0, 0)),
            pl.BlockSpec((F, D), lambda i: (0, 0)),
        ],
        out_specs=pl.BlockSpec((bm, D), lambda i: (i, 0)),
        out_shape=jax.ShapeDtypeStruct((S, D), jnp.float32),
    )(x2, gate_s, up_s, down_s)


def _add_body(ysh_ref, yg_ref, o_ref):
    y3 = yg_ref[...].reshape(ysh_ref.shape[0], 2, D)
    o_ref[...] = ysh_ref[...] + y3[:, 0, :] + y3[:, 1, :]


def _final_add(ysh, yg):
    bm = 512
    return pl.pallas_call(
        _add_body,
        grid=(S // bm,),
        in_specs=[
            pl.BlockSpec((bm, D), lambda i: (i, 0)),
            pl.BlockSpec((2 * bm, D), lambda i: (i, 0)),
        ],
        out_specs=pl.BlockSpec((bm, D), lambda i: (i, 0)),
        out_shape=jax.ShapeDtypeStruct((S, D), jnp.float32),
    )(ysh, yg)


@functools.cache
def _make_sc_gather(n_rows, n_idx):
    per_w = n_idx // _NW
    ch = 64
    n_ch = per_w // ch
    assert per_w % ch == 0
    mesh = plsc.VectorSubcoreMesh(core_axis_name="c", subcore_axis_name="s")

    @functools.partial(
        pl.kernel,
        mesh=mesh,
        out_type=jax.ShapeDtypeStruct((n_idx, D), jnp.float32),
        scratch_types=[
            pltpu.VMEM((ch,), jnp.int32),
            pltpu.VMEM((ch, D), jnp.float32),
            pltpu.SemaphoreType.DMA,
        ],
    )
    def k(table_hbm, idx_hbm, out_hbm, idx_v, rows_v, sem):
        wid = lax.axis_index("s") * _NC + lax.axis_index("c")
        base = wid * per_w
        for c in range(n_ch):
            pltpu.sync_copy(idx_hbm.at[pl.ds(base + c * ch, ch)], idx_v)
            pltpu.async_copy(table_hbm.at[idx_v], rows_v, sem).wait()
            pltpu.sync_copy(rows_v, out_hbm.at[pl.ds(base + c * ch, ch)])

    return k


def kernel(x, gate_s, up_s, down_s, Wg, Wu, Wd, Wr, br, rb):
    x2 = x.reshape(S, D)
    wr_pad = jnp.zeros((D, LANES), jnp.float32).at[:, :ER].set(Wr)
    b_pad = jnp.zeros((1, LANES), jnp.float32).at[0, :ER].set(br + rb)

    ei, wv = _routing(x2, wr_pad, b_pad)

    eflat = jnp.stack([ei[:, 0], ei[:, 1]], axis=1).reshape(A)
    wflat = jnp.stack([wv[:, 0], wv[:, 1]], axis=1).reshape(A)
    oh = (eflat[:, None] == jnp.arange(ER, dtype=jnp.int32)[None, :]).astype(jnp.int32)
    pref = jnp.cumsum(oh, axis=0)
    counts = pref[-1]
    rank = jnp.take_along_axis(pref, eflat[:, None], axis=1)[:, 0] - 1
    nblk = (counts + BM - 1) // BM
    blk_off = jnp.concatenate(
        [jnp.zeros(1, jnp.int32), jnp.cumsum(nblk).astype(jnp.int32)])
    row = blk_off[eflat] * BM + rank
    nb_real = blk_off[ER]
    tok = jnp.arange(A, dtype=jnp.int32) // TOPK
    pad_init = jnp.stack(
        [lax.bitcast_convert_type(jnp.arange(M_PAD, dtype=jnp.int32) % S,
                                  jnp.float32),
         jnp.zeros((M_PAD,), jnp.float32)], axis=1)
    packed = jnp.stack(
        [lax.bitcast_convert_type(tok, jnp.float32), wflat], axis=1)
    padded = pad_init.at[row].set(packed)
    idx_pad = lax.bitcast_convert_type(padded[:, 0], jnp.int32)
    w_pad = padded[:, 1]
    be_raw = jnp.searchsorted(blk_off[1:], jnp.arange(NB, dtype=jnp.int32),
                              side='right').astype(jnp.int32)
    last_e = jnp.take(be_raw, jnp.maximum(nb_real - 1, 0))
    be = jnp.where(jnp.arange(NB) < nb_real,
                   jnp.minimum(be_raw, ER - 1), last_e)
    scalars = jnp.concatenate([be, nb_real[None]]).astype(jnp.int32)

    ysh = _shared(x2, gate_s, up_s, down_s)
    xg = _make_sc_gather(S, M_PAD)(x2, idx_pad)
    w2d = jnp.broadcast_to(w_pad[:, None], (M_PAD, LANES))
    yr = _grouped(scalars, xg, Wg, Wu, Wd, w2d)
    yg = _make_sc_gather(M_PAD, A)(yr, row)
    out = _final_add(ysh, yg)
    return out.reshape(1, S, D)


# speedup vs baseline: 1.7364x; 1.7364x over previous
import functools

import jax
import jax.numpy as jnp
from jax import lax
from jax.experimental import pallas as pl
from jax.experimental.pallas import tpu as pltpu
from jax.experimental.pallas import tpu_sc as plsc

S, D, F, ER, TOPK = 2048, 1024, 1024, 7, 2
A = S * TOPK
BM = 256
NB = 24
M_PAD = NB * BM
LANES = 128

_NC, _NS = 2, 16
_NW = _NC * _NS


def _routing_body(x_ref, wr_ref, b_ref, ei_ref, wv_ref):
    x = x_ref[...]
    logits = jnp.dot(x, wr_ref[...], preferred_element_type=jnp.float32)
    logits = logits + b_ref[...]
    lane = lax.broadcasted_iota(jnp.int32, (S, LANES), 1)
    probs = jax.nn.sigmoid(logits)
    probs = jnp.where(lane < ER, probs, -1.0)
    m1 = jnp.max(probs, axis=1, keepdims=True)
    i1 = jnp.min(jnp.where(probs == m1, lane, LANES), axis=1, keepdims=True)
    probs2 = jnp.where(lane == i1, -1.0, probs)
    m2 = jnp.max(probs2, axis=1, keepdims=True)
    i2 = jnp.min(jnp.where(probs2 == m2, lane, LANES), axis=1, keepdims=True)
    ssum = m1 + m2
    ei_ref[...] = jnp.where(lane == 0, i1, jnp.where(lane == 1, i2, 0))
    wv_ref[...] = jnp.where(lane == 0, m1 / ssum,
                            jnp.where(lane == 1, m2 / ssum, 0.0))


def _routing(x2, wr_pad, b_pad):
    return pl.pallas_call(
        _routing_body,
        out_shape=(
            jax.ShapeDtypeStruct((S, LANES), jnp.int32),
            jax.ShapeDtypeStruct((S, LANES), jnp.float32),
        ),
    )(x2, wr_pad, b_pad)


def _grouped_body(sc_ref, xg_ref, wg_ref, wu_ref, wd_ref, wp_ref, y_ref):
    i = pl.program_id(0)

    @pl.when(i < sc_ref[NB])
    def _():
        x = xg_ref[...].astype(jnp.bfloat16)
        g = jnp.dot(x, wg_ref[0].astype(jnp.bfloat16),
                    preferred_element_type=jnp.float32)
        u = jnp.dot(x, wu_ref[0].astype(jnp.bfloat16),
                    preferred_element_type=jnp.float32)
        h = g * jax.nn.sigmoid(g) * u * wp_ref[:, :1]
        y_ref[...] = jnp.dot(h.astype(jnp.bfloat16),
                             wd_ref[0].astype(jnp.bfloat16),
                             preferred_element_type=jnp.float32)


def _grouped(scalars, xg, Wg, Wu, Wd, w2d):
    grid_spec = pltpu.PrefetchScalarGridSpec(
        num_scalar_prefetch=1,
        grid=(NB,),
        in_specs=[
            pl.BlockSpec((BM, D), lambda i, sc: (i, 0)),
            pl.BlockSpec((1, D, F), lambda i, sc: (sc[i], 0, 0)),
            pl.BlockSpec((1, D, F), lambda i, sc: (sc[i], 0, 0)),
            pl.BlockSpec((1, F, D), lambda i, sc: (sc[i], 0, 0)),
            pl.BlockSpec((BM, LANES), lambda i, sc: (i, 0)),
        ],
        out_specs=pl.BlockSpec((BM, D), lambda i, sc: (i, 0)),
    )
    return pl.pallas_call(
        _grouped_body,
        grid_spec=grid_spec,
        out_shape=jax.ShapeDtypeStruct((M_PAD, D), jnp.float32),
    )(scalars, xg, Wg, Wu, Wd, w2d)


def _shared_body(x_ref, g_ref, u_ref, d_ref, o_ref):
    x = x_ref[...].astype(jnp.bfloat16)
    g = jnp.dot(x, g_ref[...].astype(jnp.bfloat16),
                preferred_element_type=jnp.float32)
    u = jnp.dot(x, u_ref[...].astype(jnp.bfloat16),
                preferred_element_type=jnp.float32)
    h = g * jax.nn.sigmoid(g) * u
    o_ref[...] = jnp.dot(h.astype(jnp.bfloat16), d_ref[...].astype(jnp.bfloat16),
                         preferred_element_type=jnp.float32)


def _shared(x2, gate_s, up_s, down_s):
    bm = 256
    return pl.pallas_call(
        _shared_body,
        grid=(S // bm,),
        in_specs=[
            pl.BlockSpec((bm, D), lambda i: (i, 0)),
            pl.BlockSpec((D, F), lambda i: (0, 0)),
            pl.BlockSpec((D, F), lambda i: (0, 0)),
            pl.BlockSpec((F, D), lambda i: (0, 0)),
        ],
        out_specs=pl.BlockSpec((bm, D), lambda i: (i, 0)),
        out_shape=jax.ShapeDtypeStruct((S, D), jnp.float32),
    )(x2, gate_s, up_s, down_s)


def _add_body(ysh_ref, yg_ref, o_ref):
    o_ref[...] = ysh_ref[...] + yg_ref[:, :D] + yg_ref[:, D:]


def _final_add(ysh, yg):
    bm = 512
    yg2 = yg.reshape(S, 2 * D)
    return pl.pallas_call(
        _add_body,
        grid=(S // bm,),
        in_specs=[
            pl.BlockSpec((bm, D), lambda i: (i, 0)),
            pl.BlockSpec((bm, 2 * D), lambda i: (i, 0)),
        ],
        out_specs=pl.BlockSpec((bm, D), lambda i: (i, 0)),
        out_shape=jax.ShapeDtypeStruct((S, D), jnp.float32),
    )(ysh, yg2)


@functools.cache
def _make_sc_gather(n_rows, n_idx):
    per_w = n_idx // _NW
    ch = 64
    n_ch = per_w // ch
    assert per_w % ch == 0
    mesh = plsc.VectorSubcoreMesh(core_axis_name="c", subcore_axis_name="s")

    @functools.partial(
        pl.kernel,
        mesh=mesh,
        out_type=jax.ShapeDtypeStruct((n_idx, D), jnp.float32),
        scratch_types=[
            pltpu.VMEM((ch,), jnp.int32),
            pltpu.VMEM((ch, D), jnp.float32),
            pltpu.SemaphoreType.DMA,
        ],
    )
    def k(table_hbm, idx_hbm, out_hbm, idx_v, rows_v, sem):
        wid = lax.axis_index("s") * _NC + lax.axis_index("c")
        base = wid * per_w
        for c in range(n_ch):
            pltpu.sync_copy(idx_hbm.at[pl.ds(base + c * ch, ch)], idx_v)
            pltpu.async_copy(table_hbm.at[idx_v], rows_v, sem).wait()
            pltpu.sync_copy(rows_v, out_hbm.at[pl.ds(base + c * ch, ch)])

    return k


def kernel(x, gate_s, up_s, down_s, Wg, Wu, Wd, Wr, br, rb):
    x2 = x.reshape(S, D)
    wr_pad = jnp.zeros((D, LANES), jnp.float32).at[:, :ER].set(Wr)
    b_pad = jnp.zeros((1, LANES), jnp.float32).at[0, :ER].set(br + rb)

    ei, wv = _routing(x2, wr_pad, b_pad)

    eflat = jnp.stack([ei[:, 0], ei[:, 1]], axis=1).reshape(A)
    wflat = jnp.stack([wv[:, 0], wv[:, 1]], axis=1).reshape(A)
    oh = (eflat[:, None] == jnp.arange(ER, dtype=jnp.int32)[None, :]).astype(jnp.int32)
    pref = jnp.cumsum(oh, axis=0)
    counts = pref[-1]
    rank = jnp.take_along_axis(pref, eflat[:, None], axis=1)[:, 0] - 1
    nblk = (counts + BM - 1) // BM
    blk_off = jnp.concatenate(
        [jnp.zeros(1, jnp.int32), jnp.cumsum(nblk).astype(jnp.int32)])
    row = blk_off[eflat] * BM + rank
    nb_real = blk_off[ER]
    tok = jnp.arange(A, dtype=jnp.int32) // TOPK
    pad_init = jnp.stack(
        [jnp.arange(M_PAD, dtype=jnp.int32) % S,
         jnp.zeros((M_PAD,), jnp.int32)], axis=1)
    packed = jnp.stack(
        [tok, lax.bitcast_convert_type(wflat, jnp.int32)], axis=1)
    padded = pad_init.at[row].set(packed)
    idx_pad = padded[:, 0]
    w_pad = lax.bitcast_convert_type(padded[:, 1], jnp.float32)
    be_raw = jnp.searchsorted(blk_off[1:], jnp.arange(NB, dtype=jnp.int32),
                              side='right').astype(jnp.int32)
    last_e = jnp.take(be_raw, jnp.maximum(nb_real - 1, 0))
    be = jnp.where(jnp.arange(NB) < nb_real,
                   jnp.minimum(be_raw, ER - 1), last_e)
    scalars = jnp.concatenate([be, nb_real[None]]).astype(jnp.int32)

    ysh = _shared(x2, gate_s, up_s, down_s)
    xg = _make_sc_gather(S, M_PAD)(x2, idx_pad)
    w2d = jnp.broadcast_to(w_pad[:, None], (M_PAD, LANES))
    yr = _grouped(scalars, xg, Wg, Wu, Wd, w2d)
    yg = _make_sc_gather(M_PAD, A)(yr, row)
    out = _final_add(ysh, yg)
    return out.reshape(1, S, D)


# speedup vs baseline: 2.0292x; 1.1686x over previous
import functools

import jax
import jax.numpy as jnp
from jax import lax
from jax.experimental import pallas as pl
from jax.experimental.pallas import tpu as pltpu
from jax.experimental.pallas import tpu_sc as plsc

S, D, F, ER, TOPK = 2048, 1024, 1024, 7, 2
A = S * TOPK
BM = 256
NB = 24
M_PAD = NB * BM
LANES = 128

_NC, _NS = 2, 16
_NW = _NC * _NS


def _routing_body(x_ref, wr_ref, b_ref, ei_ref, wv_ref):
    x = x_ref[...]
    logits = jnp.dot(x, wr_ref[...], preferred_element_type=jnp.float32)
    logits = logits + b_ref[...]
    lane = lax.broadcasted_iota(jnp.int32, (S, LANES), 1)
    probs = jax.nn.sigmoid(logits)
    probs = jnp.where(lane < ER, probs, -1.0)
    m1 = jnp.max(probs, axis=1, keepdims=True)
    i1 = jnp.min(jnp.where(probs == m1, lane, LANES), axis=1, keepdims=True)
    probs2 = jnp.where(lane == i1, -1.0, probs)
    m2 = jnp.max(probs2, axis=1, keepdims=True)
    i2 = jnp.min(jnp.where(probs2 == m2, lane, LANES), axis=1, keepdims=True)
    ssum = m1 + m2
    ei_ref[...] = jnp.where(lane == 0, i1, jnp.where(lane == 1, i2, 0))
    wv_ref[...] = jnp.where(lane == 0, m1 / ssum,
                            jnp.where(lane == 1, m2 / ssum, 0.0))


def _routing(x2, wr_pad, b_pad):
    return pl.pallas_call(
        _routing_body,
        out_shape=(
            jax.ShapeDtypeStruct((S, LANES), jnp.int32),
            jax.ShapeDtypeStruct((S, LANES), jnp.float32),
        ),
    )(x2, wr_pad, b_pad)


def _grouped_body(sc_ref, xg_ref, wg_ref, wu_ref, wd_ref, y_ref):
    i = pl.program_id(0)

    @pl.when(i < sc_ref[NB])
    def _():
        x = xg_ref[...].astype(jnp.bfloat16)
        g = jnp.dot(x, wg_ref[0].astype(jnp.bfloat16),
                    preferred_element_type=jnp.float32)
        u = jnp.dot(x, wu_ref[0].astype(jnp.bfloat16),
                    preferred_element_type=jnp.float32)
        h = g * jax.nn.sigmoid(g) * u
        y_ref[...] = jnp.dot(h.astype(jnp.bfloat16),
                             wd_ref[0].astype(jnp.bfloat16),
                             preferred_element_type=jnp.float32)


def _grouped(scalars, xg, Wg, Wu, Wd):
    grid_spec = pltpu.PrefetchScalarGridSpec(
        num_scalar_prefetch=1,
        grid=(NB,),
        in_specs=[
            pl.BlockSpec((BM, D), lambda i, sc: (i, 0)),
            pl.BlockSpec((1, D, F), lambda i, sc: (sc[i], 0, 0)),
            pl.BlockSpec((1, D, F), lambda i, sc: (sc[i], 0, 0)),
            pl.BlockSpec((1, F, D), lambda i, sc: (sc[i], 0, 0)),
        ],
        out_specs=pl.BlockSpec((BM, D), lambda i, sc: (i, 0)),
    )
    return pl.pallas_call(
        _grouped_body,
        grid_spec=grid_spec,
        out_shape=jax.ShapeDtypeStruct((M_PAD, D), jnp.float32),
    )(scalars, xg, Wg, Wu, Wd)


def _shared_body(x_ref, g_ref, u_ref, d_ref, o_ref):
    x = x_ref[...].astype(jnp.bfloat16)
    g = jnp.dot(x, g_ref[...].astype(jnp.bfloat16),
                preferred_element_type=jnp.float32)
    u = jnp.dot(x, u_ref[...].astype(jnp.bfloat16),
                preferred_element_type=jnp.float32)
    h = g * jax.nn.sigmoid(g) * u
    o_ref[...] = jnp.dot(h.astype(jnp.bfloat16), d_ref[...].astype(jnp.bfloat16),
                         preferred_element_type=jnp.float32)


def _shared(x2, gate_s, up_s, down_s):
    bm = 256
    return pl.pallas_call(
        _shared_body,
        grid=(S // bm,),
        in_specs=[
            pl.BlockSpec((bm, D), lambda i: (i, 0)),
            pl.BlockSpec((D, F), lambda i: (0, 0)),
            pl.BlockSpec((D, F), lambda i: (0, 0)),
            pl.BlockSpec((F, D), lambda i: (0, 0)),
        ],
        out_specs=pl.BlockSpec((bm, D), lambda i: (i, 0)),
        out_shape=jax.ShapeDtypeStruct((S, D), jnp.float32),
    )(x2, gate_s, up_s, down_s)


def _add_body(ysh_ref, wv_ref, yg_ref, o_ref):
    o_ref[...] = (ysh_ref[...]
                  + wv_ref[:, :1] * yg_ref[:, :D]
                  + wv_ref[:, 1:2] * yg_ref[:, D:])


def _final_add(ysh, wv, yg):
    bm = 512
    yg2 = yg.reshape(S, 2 * D)
    return pl.pallas_call(
        _add_body,
        grid=(S // bm,),
        in_specs=[
            pl.BlockSpec((bm, D), lambda i: (i, 0)),
            pl.BlockSpec((bm, LANES), lambda i: (i, 0)),
            pl.BlockSpec((bm, 2 * D), lambda i: (i, 0)),
        ],
        out_specs=pl.BlockSpec((bm, D), lambda i: (i, 0)),
        out_shape=jax.ShapeDtypeStruct((S, D), jnp.float32),
    )(ysh, wv, yg2)


@functools.cache
def _make_sc_gather(n_rows, n_idx):
    per_w = n_idx // _NW
    ch = 64
    n_ch = per_w // ch
    assert per_w % ch == 0
    mesh = plsc.VectorSubcoreMesh(core_axis_name="c", subcore_axis_name="s")

    @functools.partial(
        pl.kernel,
        mesh=mesh,
        out_type=jax.ShapeDtypeStruct((n_idx, D), jnp.float32),
        scratch_types=[
            pltpu.VMEM((ch,), jnp.int32),
            pltpu.VMEM((ch, D), jnp.float32),
            pltpu.SemaphoreType.DMA,
        ],
    )
    def k(table_hbm, idx_hbm, out_hbm, idx_v, rows_v, sem):
        wid = lax.axis_index("s") * _NC + lax.axis_index("c")
        base = wid * per_w
        for c in range(n_ch):
            pltpu.sync_copy(idx_hbm.at[pl.ds(base + c * ch, ch)], idx_v)
            pltpu.async_copy(table_hbm.at[idx_v], rows_v, sem).wait()
            pltpu.sync_copy(rows_v, out_hbm.at[pl.ds(base + c * ch, ch)])

    return k


@functools.cache
def _make_sc_dispatch():
    per_w = S // _NW
    mesh = plsc.VectorSubcoreMesh(core_axis_name="c", subcore_axis_name="s")

    @functools.partial(
        pl.kernel,
        mesh=mesh,
        out_type=jax.ShapeDtypeStruct((M_PAD, D), jnp.float32),
        scratch_types=[
            pltpu.VMEM((per_w,), jnp.int32),
            pltpu.VMEM((per_w,), jnp.int32),
            pltpu.VMEM((per_w, D), jnp.float32),
            pltpu.SemaphoreType.DMA,
        ],
    )
    def k(x_hbm, p1_hbm, p2_hbm, xg_hbm, i1_v, i2_v, xbuf, sem):
        wid = lax.axis_index("s") * _NC + lax.axis_index("c")
        base = wid * per_w
        pltpu.sync_copy(p1_hbm.at[pl.ds(base, per_w)], i1_v)
        pltpu.sync_copy(p2_hbm.at[pl.ds(base, per_w)], i2_v)
        pltpu.sync_copy(x_hbm.at[pl.ds(base, per_w)], xbuf)
        c1 = pltpu.async_copy(xbuf, xg_hbm.at[i1_v], sem)
        c2 = pltpu.async_copy(xbuf, xg_hbm.at[i2_v], sem)
        c1.wait()
        c2.wait()

    return k


def kernel(x, gate_s, up_s, down_s, Wg, Wu, Wd, Wr, br, rb):
    x2 = x.reshape(S, D)
    wr_pad = jnp.zeros((D, LANES), jnp.float32).at[:, :ER].set(Wr)
    b_pad = jnp.zeros((1, LANES), jnp.float32).at[0, :ER].set(br + rb)

    ei, wv = _routing(x2, wr_pad, b_pad)

    eflat = jnp.stack([ei[:, 0], ei[:, 1]], axis=1).reshape(A)
    wflat = jnp.stack([wv[:, 0], wv[:, 1]], axis=1).reshape(A)
    oh = (eflat[:, None] == jnp.arange(ER, dtype=jnp.int32)[None, :]).astype(jnp.int32)
    pref = jnp.cumsum(oh, axis=0)
    counts = pref[-1]
    rank = jnp.take_along_axis(pref, eflat[:, None], axis=1)[:, 0] - 1
    nblk = (counts + BM - 1) // BM
    blk_off = jnp.concatenate(
        [jnp.zeros(1, jnp.int32), jnp.cumsum(nblk).astype(jnp.int32)])
    row = blk_off[eflat] * BM + rank
    nb_real = blk_off[ER]
    be_raw = jnp.searchsorted(blk_off[1:], jnp.arange(NB, dtype=jnp.int32),
                              side='right').astype(jnp.int32)
    last_e = jnp.take(be_raw, jnp.maximum(nb_real - 1, 0))
    be = jnp.where(jnp.arange(NB) < nb_real,
                   jnp.minimum(be_raw, ER - 1), last_e)
    scalars = jnp.concatenate([be, nb_real[None]]).astype(jnp.int32)

    ysh = _shared(x2, gate_s, up_s, down_s)
    p12 = row.reshape(S, TOPK)
    xg = _make_sc_dispatch()(x2, p12[:, 0], p12[:, 1])
    yr = _grouped(scalars, xg, Wg, Wu, Wd)
    yg = _make_sc_gather(M_PAD, A)(yr, row)
    out = _final_add(ysh, wv, yg)
    return out.reshape(1, S, D)
